# Initial kernel scaffold; baseline (speedup 1.0000x reference)
import functools

import jax
import jax.numpy as jnp
from jax import lax
from jax.experimental import pallas as pl
from jax.experimental.pallas import tpu as pltpu
from jax.experimental.pallas import tpu_sc as plsc

N = 10000
E = 320000
D = 128
DEGW = 16
NC = 2
NS = 16
NW = NC * NS
EPW = E // NW
CHUNK = 80
NCHUNK = EPW // CHUNK
RPT = N // NS
ZROWS = 125
BM = 1000
NB = N // BM


def _mesh():
    return plsc.VectorSubcoreMesh(core_axis_name="c", subcore_axis_name="s")


@functools.partial(
    pl.kernel,
    out_type=jax.ShapeDtypeStruct((NC * N, DEGW), jnp.float32),
    mesh=_mesh(),
    scratch_types=[
        pltpu.VMEM((CHUNK,), jnp.int32),
        pltpu.VMEM((CHUNK, DEGW), jnp.float32),
        pltpu.VMEM((ZROWS, DEGW), jnp.float32),
        pltpu.VMEM_SHARED((N, DEGW), jnp.float32),
    ],
)
def _deg_kernel(col_hbm, degp_hbm, idx_v, ones_v, zbuf_v, acc_sh):
    c = lax.axis_index("c")
    s = lax.axis_index("s")
    wid = s * NC + c

    zeros_row = jnp.zeros((DEGW,), jnp.float32)
    ones_row = jnp.ones((DEGW,), jnp.float32)

    def fill_z(i, carry):
        zbuf_v[i, :] = zeros_row
        return carry

    lax.fori_loop(0, ZROWS, fill_z, 0)

    def fill_o(i, carry):
        ones_v[i, :] = ones_row
        return carry

    lax.fori_loop(0, CHUNK, fill_o, 0)

    r0 = s * RPT
    for i in range(RPT // ZROWS):
        pltpu.sync_copy(zbuf_v, acc_sh.at[pl.ds(r0 + i * ZROWS, ZROWS)])
    plsc.subcore_barrier()

    base = wid * EPW

    def chunk(j, carry):
        pltpu.sync_copy(col_hbm.at[pl.ds(base + j * CHUNK, CHUNK)], idx_v)
        pltpu.sync_copy(ones_v, acc_sh.at[idx_v], add=True)
        return carry

    lax.fori_loop(0, NCHUNK, chunk, 0)
    plsc.subcore_barrier()

    pltpu.sync_copy(acc_sh.at[pl.ds(r0, RPT)],
                    degp_hbm.at[pl.ds(c * N + r0, RPT)])


@functools.partial(
    pl.kernel,
    out_type=jax.ShapeDtypeStruct((NC * N, D), jnp.float32),
    mesh=_mesh(),
    scratch_types=[
        pltpu.VMEM((CHUNK,), jnp.int32),
        pltpu.VMEM((CHUNK,), jnp.int32),
        pltpu.VMEM((CHUNK, D), jnp.float32),
        pltpu.VMEM((ZROWS, D), jnp.float32),
        pltpu.VMEM_SHARED((N, D), jnp.float32),
        pltpu.SemaphoreType.DMA,
    ],
)
def _agg_kernel(y_hbm, row_hbm, col_hbm, aggp_hbm,
                ridx_v, cidx_v, rows_v, zbuf_v, acc_sh, sem):
    c = lax.axis_index("c")
    s = lax.axis_index("s")
    wid = s * NC + c

    zeros16 = jnp.zeros((16,), jnp.float32)

    def fill_z(i, carry):
        for j in range(D // 16):
            zbuf_v[i, pl.ds(j * 16, 16)] = zeros16
        return carry

    lax.fori_loop(0, ZROWS, fill_z, 0)

    r0 = s * RPT
    for i in range(RPT // ZROWS):
        pltpu.sync_copy(zbuf_v, acc_sh.at[pl.ds(r0 + i * ZROWS, ZROWS)])
    plsc.subcore_barrier()

    base = wid * EPW

    def chunk(j, carry):
        off = base + j * CHUNK
        pltpu.sync_copy(row_hbm.at[pl.ds(off, CHUNK)], ridx_v)
        pltpu.sync_copy(col_hbm.at[pl.ds(off, CHUNK)], cidx_v)
        pltpu.async_copy(y_hbm.at[ridx_v], rows_v, sem).wait()
        pltpu.sync_copy(rows_v, acc_sh.at[cidx_v], add=True)
        return carry

    lax.fori_loop(0, NCHUNK, chunk, 0)
    plsc.subcore_barrier()

    pltpu.sync_copy(acc_sh.at[pl.ds(r0, RPT)],
                    aggp_hbm.at[pl.ds(c * N + r0, RPT)])


def _dis(d0_ref, d1_ref):
    return lax.rsqrt(1.0 + d0_ref[:, 0:1] + d1_ref[:, 0:1])


def _mm_scale_body(x_ref, w_ref, d0_ref, d1_ref, y_ref):
    dis = _dis(d0_ref, d1_ref)
    y_ref[...] = jnp.dot(x_ref[...], w_ref[...],
                         preferred_element_type=jnp.float32) * dis


def _tc_layer1(x, W1, degp):
    return pl.pallas_call(
        _mm_scale_body,
        grid=(NB,),
        in_specs=[
            pl.BlockSpec((BM, D), lambda i: (i, 0)),
            pl.BlockSpec((D, D), lambda i: (0, 0)),
            pl.BlockSpec((BM, DEGW), lambda i: (i, 0)),
            pl.BlockSpec((BM, DEGW), lambda i: (NB + i, 0)),
        ],
        out_specs=pl.BlockSpec((BM, D), lambda i: (i, 0)),
        out_shape=jax.ShapeDtypeStruct((N, D), jnp.float32),
    )(x, W1, degp, degp)


def _combine_mm_body(a0_ref, a1_ref, y1_ref, d0_ref, d1_ref, w_ref, b_ref,
                     y2_ref):
    dis = _dis(d0_ref, d1_ref)
    h = jnp.maximum(
        dis * (a0_ref[...] + a1_ref[...] + y1_ref[...]) + b_ref[...], 0.0)
    y2_ref[...] = jnp.dot(h, w_ref[...],
                          preferred_element_type=jnp.float32) * dis


def _tc_layer2(aggp, y1, degp, W2, b1):
    return pl.pallas_call(
        _combine_mm_body,
        grid=(NB,),
        in_specs=[
            pl.BlockSpec((BM, D), lambda i: (i, 0)),
            pl.BlockSpec((BM, D), lambda i: (NB + i, 0)),
            pl.BlockSpec((BM, D), lambda i: (i, 0)),
            pl.BlockSpec((BM, DEGW), lambda i: (i, 0)),
            pl.BlockSpec((BM, DEGW), lambda i: (NB + i, 0)),
            pl.BlockSpec((D, D), lambda i: (0, 0)),
            pl.BlockSpec((1, D), lambda i: (0, 0)),
        ],
        out_specs=pl.BlockSpec((BM, D), lambda i: (i, 0)),
        out_shape=jax.ShapeDtypeStruct((N, D), jnp.float32),
    )(aggp, aggp, y1, degp, degp, W2, b1)


def _final_body(a0_ref, a1_ref, y2_ref, d0_ref, d1_ref, b_ref, o_ref):
    dis = _dis(d0_ref, d1_ref)
    o_ref[...] = jnp.maximum(
        dis * (a0_ref[...] + a1_ref[...] + y2_ref[...]) + b_ref[...], 0.0)


def _tc_final(aggp, y2, degp, b2):
    return pl.pallas_call(
        _final_body,
        grid=(NB,),
        in_specs=[
            pl.BlockSpec((BM, D), lambda i: (i, 0)),
            pl.BlockSpec((BM, D), lambda i: (NB + i, 0)),
            pl.BlockSpec((BM, D), lambda i: (i, 0)),
            pl.BlockSpec((BM, DEGW), lambda i: (i, 0)),
            pl.BlockSpec((BM, DEGW), lambda i: (NB + i, 0)),
            pl.BlockSpec((1, D), lambda i: (0, 0)),
        ],
        out_specs=pl.BlockSpec((BM, D), lambda i: (i, 0)),
        out_shape=jax.ShapeDtypeStruct((N, D), jnp.float32),
    )(aggp, aggp, y2, degp, degp, b2)


def kernel(x, edge_index, W1, b1, W2, b2):
    ei = edge_index.astype(jnp.int32)
    row = ei[0]
    col = ei[1]
    degp = _deg_kernel(col)
    y1 = _tc_layer1(x, W1, degp)
    a1 = _agg_kernel(y1, row, col)
    y2 = _tc_layer2(a1, y1, degp, W2, b1.reshape(1, D))
    a2 = _agg_kernel(y2, row, col)
    return _tc_final(a2, y2, degp, b2.reshape(1, D))


# speedup vs baseline: 12.1421x; 12.1421x over previous
import functools

import jax
import jax.numpy as jnp
from jax import lax
from jax.experimental import pallas as pl
from jax.experimental.pallas import tpu as pltpu
from jax.experimental.pallas import tpu_sc as plsc

N = 10000
NPAD = 10240
E = 320000
D = 128
DEGW = 128
NC = 2
NS = 16
NW = NC * NS
EPW = E // NW
CHUNK = 80
NCHUNK = EPW // CHUNK
RPT = NPAD // NS
ZROWS = 128
BM = 1000
NB = N // BM


def _mesh():
    return plsc.VectorSubcoreMesh(core_axis_name="c", subcore_axis_name="s")


@functools.partial(
    pl.kernel,
    out_type=jax.ShapeDtypeStruct((NC * NPAD, DEGW), jnp.float32),
    mesh=_mesh(),
    scratch_types=[
        pltpu.VMEM((CHUNK,), jnp.int32),
        pltpu.VMEM((CHUNK, DEGW), jnp.float32),
        pltpu.VMEM((ZROWS, DEGW), jnp.float32),
        pltpu.VMEM_SHARED((NPAD, DEGW), jnp.float32),
    ],
)
def _deg_kernel(col_hbm, degp_hbm, idx_v, ones_v, zbuf_v, acc_sh):
    c = lax.axis_index("c")
    s = lax.axis_index("s")
    wid = s * NC + c

    zeros16 = jnp.zeros((16,), jnp.float32)
    ones16 = jnp.ones((16,), jnp.float32)

    def fill_z(i, carry):
        for j in range(DEGW // 16):
            zbuf_v[i, pl.ds(j * 16, 16)] = zeros16
        return carry

    lax.fori_loop(0, ZROWS, fill_z, 0)

    def fill_o(i, carry):
        for j in range(DEGW // 16):
            ones_v[i, pl.ds(j * 16, 16)] = ones16
        return carry

    lax.fori_loop(0, CHUNK, fill_o, 0)

    r0 = s * RPT
    for i in range(RPT // ZROWS):
        pltpu.sync_copy(zbuf_v, acc_sh.at[pl.ds(r0 + i * ZROWS, ZROWS)])
    plsc.subcore_barrier()

    base = wid * EPW

    def chunk(j, carry):
        pltpu.sync_copy(col_hbm.at[pl.ds(base + j * CHUNK, CHUNK)], idx_v)
        pltpu.sync_copy(ones_v, acc_sh.at[idx_v], add=True)
        return carry

    lax.fori_loop(0, NCHUNK, chunk, 0)
    plsc.subcore_barrier()

    pltpu.sync_copy(acc_sh.at[pl.ds(r0, RPT)],
                    degp_hbm.at[pl.ds(c * NPAD + r0, RPT)])


@functools.partial(
    pl.kernel,
    out_type=jax.ShapeDtypeStruct((NC * NPAD, D), jnp.float32),
    mesh=_mesh(),
    scratch_types=[
        pltpu.VMEM((CHUNK,), jnp.int32),
        pltpu.VMEM((CHUNK,), jnp.int32),
        pltpu.VMEM((CHUNK, D), jnp.float32),
        pltpu.VMEM((ZROWS, D), jnp.float32),
        pltpu.VMEM_SHARED((NPAD, D), jnp.float32),
        pltpu.SemaphoreType.DMA,
    ],
)
def _agg_kernel(y_hbm, row_hbm, col_hbm, aggp_hbm,
                ridx_v, cidx_v, rows_v, zbuf_v, acc_sh, sem):
    c = lax.axis_index("c")
    s = lax.axis_index("s")
    wid = s * NC + c

    zeros16 = jnp.zeros((16,), jnp.float32)

    def fill_z(i, carry):
        for j in range(D // 16):
            zbuf_v[i, pl.ds(j * 16, 16)] = zeros16
        return carry

    lax.fori_loop(0, ZROWS, fill_z, 0)

    r0 = s * RPT
    for i in range(RPT // ZROWS):
        pltpu.sync_copy(zbuf_v, acc_sh.at[pl.ds(r0 + i * ZROWS, ZROWS)])
    plsc.subcore_barrier()

    base = wid * EPW

    def chunk(j, carry):
        off = base + j * CHUNK
        pltpu.sync_copy(row_hbm.at[pl.ds(off, CHUNK)], ridx_v)
        pltpu.sync_copy(col_hbm.at[pl.ds(off, CHUNK)], cidx_v)
        pltpu.async_copy(y_hbm.at[ridx_v], rows_v, sem).wait()
        pltpu.sync_copy(rows_v, acc_sh.at[cidx_v], add=True)
        return carry

    lax.fori_loop(0, NCHUNK, chunk, 0)
    plsc.subcore_barrier()

    pltpu.sync_copy(acc_sh.at[pl.ds(r0, RPT)],
                    aggp_hbm.at[pl.ds(c * NPAD + r0, RPT)])


def _dis(d0_ref, d1_ref):
    return lax.rsqrt(1.0 + d0_ref[:, 0:1] + d1_ref[:, 0:1])


def _mm_scale_body(x_ref, w_ref, d0_ref, d1_ref, y_ref):
    dis = _dis(d0_ref, d1_ref)
    y_ref[...] = jnp.dot(x_ref[...], w_ref[...],
                         preferred_element_type=jnp.float32) * dis


def _tc_layer1(x, W1, d0, d1):
    return pl.pallas_call(
        _mm_scale_body,
        grid=(NB,),
        in_specs=[
            pl.BlockSpec((BM, D), lambda i: (i, 0)),
            pl.BlockSpec((D, D), lambda i: (0, 0)),
            pl.BlockSpec((BM, DEGW), lambda i: (i, 0)),
            pl.BlockSpec((BM, DEGW), lambda i: (i, 0)),
        ],
        out_specs=pl.BlockSpec((BM, D), lambda i: (i, 0)),
        out_shape=jax.ShapeDtypeStruct((N, D), jnp.float32),
    )(x, W1, d0, d1)


def _combine_mm_body(a0_ref, a1_ref, y1_ref, d0_ref, d1_ref, w_ref, b_ref,
                     y2_ref):
    dis = _dis(d0_ref, d1_ref)
    h = jnp.maximum(
        dis * (a0_ref[...] + a1_ref[...] + y1_ref[...]) + b_ref[...], 0.0)
    y2_ref[...] = jnp.dot(h, w_ref[...],
                          preferred_element_type=jnp.float32) * dis


def _tc_layer2(a0, a1, y1, d0, d1, W2, b1):
    return pl.pallas_call(
        _combine_mm_body,
        grid=(NB,),
        in_specs=[
            pl.BlockSpec((BM, D), lambda i: (i, 0)),
            pl.BlockSpec((BM, D), lambda i: (i, 0)),
            pl.BlockSpec((BM, D), lambda i: (i, 0)),
            pl.BlockSpec((BM, DEGW), lambda i: (i, 0)),
            pl.BlockSpec((BM, DEGW), lambda i: (i, 0)),
            pl.BlockSpec((D, D), lambda i: (0, 0)),
            pl.BlockSpec((1, D), lambda i: (0, 0)),
        ],
        out_specs=pl.BlockSpec((BM, D), lambda i: (i, 0)),
        out_shape=jax.ShapeDtypeStruct((N, D), jnp.float32),
    )(a0, a1, y1, d0, d1, W2, b1)


def _final_body(a0_ref, a1_ref, y2_ref, d0_ref, d1_ref, b_ref, o_ref):
    dis = _dis(d0_ref, d1_ref)
    o_ref[...] = jnp.maximum(
        dis * (a0_ref[...] + a1_ref[...] + y2_ref[...]) + b_ref[...], 0.0)


def _tc_final(a0, a1, y2, d0, d1, b2):
    return pl.pallas_call(
        _final_body,
        grid=(NB,),
        in_specs=[
            pl.BlockSpec((BM, D), lambda i: (i, 0)),
            pl.BlockSpec((BM, D), lambda i: (i, 0)),
            pl.BlockSpec((BM, D), lambda i: (i, 0)),
            pl.BlockSpec((BM, DEGW), lambda i: (i, 0)),
            pl.BlockSpec((BM, DEGW), lambda i: (i, 0)),
            pl.BlockSpec((1, D), lambda i: (0, 0)),
        ],
        out_specs=pl.BlockSpec((BM, D), lambda i: (i, 0)),
        out_shape=jax.ShapeDtypeStruct((N, D), jnp.float32),
    )(a0, a1, y2, d0, d1, b2)


def kernel(x, edge_index, W1, b1, W2, b2):
    ei = edge_index.astype(jnp.int32)
    row = ei[0]
    col = ei[1]
    degp = _deg_kernel(col)
    d0 = degp[:N]
    d1 = degp[NPAD:NPAD + N]
    y1 = _tc_layer1(x, W1, d0, d1)
    a1 = _agg_kernel(y1, row, col)
    y2 = _tc_layer2(a1[:N], a1[NPAD:NPAD + N], y1, d0, d1,
                    W2, b1.reshape(1, D))
    a2 = _agg_kernel(y2, row, col)
    return _tc_final(a2[:N], a2[NPAD:NPAD + N], y2, d0, d1,
                     b2.reshape(1, D))
